# baseline (speedup 1.0000x reference)
import functools

import jax
import jax.numpy as jnp
from jax import lax
from jax.experimental import pallas as pl
from jax.experimental.pallas import tpu as pltpu
from jax.experimental.pallas import tpu_sc as plsc

_VOCAB = 1000000
_EMB = 64
_BATCH = 4096
_SEQ = 200
_CLS = 5

_NC = 2
_NS = 16
_NW = _NC * _NS
_BPW = _BATCH // _NW
_LANES = 16
_CHUNKS = _EMB // _LANES


def _pool_body(inputs_hbm, table_hbm, out_hbm, idx_v, rows_v, pooled_v,
               sem0, sem1):
    wid = lax.axis_index("s") * _NC + lax.axis_index("c")
    base = wid * _BPW

    pltpu.sync_copy(inputs_hbm.at[pl.ds(base, _BPW)], idx_v)

    def gather_start(i, buf, sem):
        pltpu.async_copy(table_hbm.at[idx_v.at[i]], rows_v.at[buf], sem)

    def gather_wait(i, buf, sem):
        pltpu.make_async_copy(table_hbm.at[idx_v.at[i]], rows_v.at[buf],
                              sem).wait()

    def accumulate(i, buf):
        rows = rows_v.at[buf]

        def tstep(t, acc):
            return tuple(
                acc[c] + rows[t, pl.ds(c * _LANES, _LANES)]
                for c in range(_CHUNKS)
            )

        acc = lax.fori_loop(
            0, _SEQ, tstep,
            tuple(jnp.zeros((_LANES,), jnp.float32) for _ in range(_CHUNKS)),
            unroll=2,
        )
        inv = jnp.float32(1.0 / _SEQ)
        for c in range(_CHUNKS):
            pooled_v[i, pl.ds(c * _LANES, _LANES)] = acc[c] * inv

    gather_start(0, 0, sem0)

    def outer(g, _):
        i0 = g * 2
        i1 = i0 + 1
        gather_start(i1, 1, sem1)
        gather_wait(i0, 0, sem0)
        accumulate(i0, 0)

        @pl.when(i1 + 1 < _BPW)
        def _():
            gather_start(i1 + 1, 0, sem0)

        gather_wait(i1, 1, sem1)
        accumulate(i1, 1)
        return 0

    lax.fori_loop(0, _BPW // 2, outer, 0)

    pltpu.sync_copy(pooled_v, out_hbm.at[pl.ds(base, _BPW)])


_pool = functools.partial(
    pl.kernel,
    out_type=jax.ShapeDtypeStruct((_BATCH, _EMB), jnp.float32),
    mesh=plsc.VectorSubcoreMesh(core_axis_name="c", subcore_axis_name="s",
                                num_cores=_NC, num_subcores=_NS),
    scratch_types=[
        pltpu.VMEM((_BPW, _SEQ), jnp.int32),
        pltpu.VMEM((2, _SEQ, _EMB), jnp.float32),
        pltpu.VMEM((_BPW, _EMB), jnp.float32),
        pltpu.SemaphoreType.DMA,
        pltpu.SemaphoreType.DMA,
    ],
    compiler_params=pltpu.CompilerParams(use_tc_tiling_on_sc=False),
)(_pool_body)


def _linear_body(pooled_ref, wt_ref, b_ref, out_ref):
    out_ref[...] = (
        jnp.dot(pooled_ref[...], wt_ref[...],
                preferred_element_type=jnp.float32)
        + b_ref[...]
    )


def _linear(pooled, wt, b2):
    return pl.pallas_call(
        _linear_body,
        out_shape=jax.ShapeDtypeStruct((_BATCH, _CLS), jnp.float32),
    )(pooled, wt, b2)


def kernel(inputs, emb_table, W, b):
    pooled = _pool(inputs, emb_table)
    return _linear(pooled, W.T, b[None, :])
